# baseline (speedup 1.0000x reference)
import functools
import math

import jax
import jax.numpy as jnp
from jax import lax
from jax.experimental import pallas as pl
from jax.experimental.pallas import tpu as pltpu
from jax.experimental.pallas import tpu_sc as plsc

D_MODEL = 64
SCALE = math.sqrt(D_MODEL)
NC = 2
NS = 16
NW = NC * NS
L = 16
NBLK = 128
NB = 4


PCH = 80


def _depad(table):
    V = table.shape[0]
    n_chunks = (V // 2) // PCH
    iters = -(-n_chunks // NW)

    mesh = plsc.VectorSubcoreMesh(core_axis_name="c", subcore_axis_name="s")

    @functools.partial(
        pl.kernel,
        mesh=mesh,
        out_type=jax.ShapeDtypeStruct((V // 2, 2 * D_MODEL), jnp.float32),
        scratch_types=[
            pltpu.VMEM((2, 2 * PCH, D_MODEL), jnp.float32),
            pltpu.VMEM((2, PCH, 2 * D_MODEL), jnp.float32),
            [pltpu.SemaphoreType.DMA] * 2,
            [pltpu.SemaphoreType.DMA] * 2,
        ],
    )
    def k(tab_hbm, out_hbm, ibufs, pbufs, isems, osems):
        wid = lax.axis_index("s") * NC + lax.axis_index("c")

        def chunk_of(ci):
            return jnp.minimum(wid + ci * NW, n_chunks - 1)

        def in_start(ci, p):
            pltpu.async_copy(
                tab_hbm.at[pl.ds(chunk_of(ci) * (2 * PCH), 2 * PCH)],
                ibufs.at[p],
                isems[p],
            )

        def in_wait(p):
            pltpu.make_async_copy(
                tab_hbm.at[pl.ds(0, 2 * PCH)], ibufs.at[p], isems[p]
            ).wait()

        def out_start(ci, p):
            pltpu.async_copy(
                pbufs.at[p],
                out_hbm.at[pl.ds(chunk_of(ci) * PCH, PCH)],
                osems[p],
            )

        def out_wait(p):
            pltpu.make_async_copy(
                pbufs.at[p], out_hbm.at[pl.ds(0, PCH)], osems[p]
            ).wait()

        in_start(0, 0)
        in_start(1, 1)

        @pl.loop(0, iters, step=2)
        def superstep(g0):
            for p in range(2):
                g = g0 + p
                in_wait(p)

                @pl.when(g >= 2)
                def _():
                    out_wait(p)

                @plsc.parallel_loop(0, 2 * PCH, unroll=4)
                def row(r):
                    half = (r & 1) * D_MODEL
                    for q in range(D_MODEL // L):
                        pbufs[p, r >> 1, pl.ds(half + q * L, L)] = ibufs[
                            p, r, pl.ds(q * L, L)
                        ]

                @pl.when(g + 2 < iters)
                def _():
                    in_start(g + 2, p)

                out_start(g, p)

        for p in range(2):
            out_wait(p)

    return k(table)


def _embed(xt, table, n_tokens, seq):
    n_blocks = n_tokens // NBLK
    assert n_blocks == NW and seq % NB == 0

    mesh = plsc.VectorSubcoreMesh(core_axis_name="c", subcore_axis_name="s")

    @functools.partial(
        pl.kernel,
        mesh=mesh,
        out_type=jax.ShapeDtypeStruct(
            (seq, D_MODEL // 8, n_blocks, 8, NBLK), jnp.float32
        ),
        compiler_params=pltpu.CompilerParams(
            use_tc_tiling_on_sc=False, needs_layout_passes=False
        ),
        scratch_types=[
            pltpu.VMEM((seq, NBLK), jnp.int32),
            pltpu.VMEM((NB, NBLK, D_MODEL), jnp.float32),
            pltpu.VMEM((NB, D_MODEL // 8, 8, NBLK + 1), jnp.float32),
            [pltpu.SemaphoreType.DMA] * NB,
            [pltpu.SemaphoreType.DMA] * NB,
        ],
    )
    def k(x_hbm, tab_hbm, out_hbm, idx_v, gbufs, obufs, gsems, ssems):
        wid = lax.axis_index("s") * NC + lax.axis_index("c")
        pltpu.sync_copy(x_hbm.at[:, pl.ds(wid * NBLK, NBLK)], idx_v)

        def gather_start(s, p):
            pltpu.async_copy(
                tab_hbm.at[idx_v.at[s]], gbufs.at[p], gsems[p]
            )

        def gather_wait(p):
            pltpu.make_async_copy(
                tab_hbm.at[idx_v.at[0]], gbufs.at[p], gsems[p]
            ).wait()

        def store_start(s, p):
            pltpu.async_copy(
                obufs.at[p, :, :, pl.ds(0, NBLK)],
                out_hbm.at[s, :, wid],
                ssems[p],
            )

        def store_wait(p):
            pltpu.make_async_copy(
                obufs.at[p, :, :, pl.ds(0, NBLK)],
                out_hbm.at[0, :, wid],
                ssems[p],
            ).wait()

        gather_start(0, 0)
        gather_start(1, 1)

        iota = jax.lax.iota(jnp.int32, L)
        dts = [(iota + q * L) >> 3 for q in range(D_MODEL // L)]
        dbs = [(iota + q * L) & 7 for q in range(D_MODEL // L)]

        @pl.loop(0, seq, step=NB)
        def superstep(s0):
            for p in range(NB):
                s = s0 + p
                gather_wait(p)

                @pl.when(s >= NB)
                def _():
                    store_wait(p)

                @plsc.parallel_loop(0, NBLK, unroll=4)
                def tok(t):
                    colt = jnp.full((L,), t, jnp.int32)
                    for q in range(D_MODEL // L):
                        v = gbufs[p, t, pl.ds(q * L, L)]
                        plsc.store_scatter(
                            obufs.at[p], [dts[q], dbs[q], colt], v * SCALE
                        )

                @pl.when(s + 2 < seq)
                def _():
                    gather_start(s + 2, (p + 2) % NB)

                store_start(s, p)

        for p in range(NB):
            store_wait(p)

    return k(xt, table)


def kernel(x, table):
    n, seq = x.shape
    packed = _depad(table)
    out5 = _embed(x.T, packed.reshape(-1, D_MODEL), n, seq)
    return out5.transpose(2, 4, 0, 1, 3).reshape(n, seq, D_MODEL)


# speedup vs baseline: 1.0375x; 1.0375x over previous
import functools
import math

import jax
import jax.numpy as jnp
from jax import lax
from jax.experimental import pallas as pl
from jax.experimental.pallas import tpu as pltpu
from jax.experimental.pallas import tpu_sc as plsc

D_MODEL = 64
SCALE = math.sqrt(D_MODEL)
NC = 2
NS = 16
NW = NC * NS
L = 16
NBLK = 128
NB = 4


def _embed(xt, table, n_tokens, seq):
    n_blocks = n_tokens // NBLK
    assert n_blocks == NW and seq % NB == 0

    mesh = plsc.VectorSubcoreMesh(core_axis_name="c", subcore_axis_name="s")

    @functools.partial(
        pl.kernel,
        mesh=mesh,
        out_type=jax.ShapeDtypeStruct(
            (seq, D_MODEL // 8, n_blocks, 8, NBLK), jnp.float32
        ),
        compiler_params=pltpu.CompilerParams(
            use_tc_tiling_on_sc=False, needs_layout_passes=False
        ),
        scratch_types=[
            pltpu.VMEM((seq, NBLK), jnp.int32),
            pltpu.VMEM((NB, NBLK, D_MODEL), jnp.float32),
            pltpu.VMEM((NB, D_MODEL // 8, 8, NBLK + 1), jnp.float32),
            [pltpu.SemaphoreType.DMA] * NB,
            [pltpu.SemaphoreType.DMA] * NB,
        ],
    )
    def k(x_hbm, tab_hbm, out_hbm, idx_v, gbufs, obufs, gsems, ssems):
        wid = lax.axis_index("s") * NC + lax.axis_index("c")
        pltpu.sync_copy(x_hbm.at[:, pl.ds(wid * NBLK, NBLK)], idx_v)

        def gather_start(s, p):
            pltpu.async_copy(
                tab_hbm.at[idx_v.at[s]], gbufs.at[p], gsems[p]
            )

        def gather_wait(p):
            pltpu.make_async_copy(
                tab_hbm.at[idx_v.at[0]], gbufs.at[p], gsems[p]
            ).wait()

        def store_start(s, p):
            pltpu.async_copy(
                obufs.at[p, :, :, pl.ds(0, NBLK)],
                out_hbm.at[s, :, wid],
                ssems[p],
            )

        def store_wait(p):
            pltpu.make_async_copy(
                obufs.at[p, :, :, pl.ds(0, NBLK)],
                out_hbm.at[0, :, wid],
                ssems[p],
            ).wait()

        gather_start(0, 0)
        gather_start(1, 1)
        gather_start(2, 2)

        iota = jax.lax.iota(jnp.int32, L)
        dts = [(iota + q * L) >> 3 for q in range(D_MODEL // L)]
        dbs = [(iota + q * L) & 7 for q in range(D_MODEL // L)]

        @pl.loop(0, seq, step=NB)
        def superstep(s0):
            for p in range(NB):
                s = s0 + p
                gather_wait(p)

                @pl.when(s >= NB)
                def _():
                    store_wait(p)

                @plsc.parallel_loop(0, NBLK, unroll=8)
                def tok(t):
                    colt = jnp.full((L,), t, jnp.int32)
                    for q in range(D_MODEL // L):
                        v = gbufs[p, t, pl.ds(q * L, L)]
                        plsc.store_scatter(
                            obufs.at[p], [dts[q], dbs[q], colt], v * SCALE
                        )

                @pl.when(s + 3 < seq)
                def _():
                    gather_start(s + 3, (p + 3) % NB)

                store_start(s, p)

        for p in range(NB):
            store_wait(p)

    return k(xt, table)


def kernel(x, table):
    n, seq = x.shape
    out5 = _embed(x.T, table, n, seq)
    return out5.transpose(2, 4, 0, 1, 3).reshape(n, seq, D_MODEL)
